# baseline (speedup 1.0000x reference)
import functools

import jax
import jax.numpy as jnp
from jax import lax
from jax.experimental import pallas as pl
from jax.experimental.pallas import tpu as pltpu
from jax.experimental.pallas import tpu_sc as plsc

N_USERS = 50000
N_ITEMS = 50000
DIM = 64
NNZ = 1600000
Q = 5
BATCH = 1024
TEMP = 0.2
T2 = 0.1
T3 = 1.0
W1 = 0.2
L2 = 1e-05

HALF = DIM // 2
NS = 16
CH = 128
JB = CH // 128
RING = 4
NNZ_PAD = 1613824
N_CHUNKS = NNZ_PAD // CH
CHUNKS_PER_TILE = N_CHUNKS // NS
NTRIP = CHUNKS_PER_TILE // RING
N_ROWS_PAD = 50048
ROWS_PT = N_ROWS_PAD // NS


def _spmm_body(tbl0, tbl1, init0, init1, ids2, vals2,
               out0, out1, acc,
               ids_r, vals_r, srows_r, gbuf_r,
               gsem_r, ssem_r, isem_r):
    c = lax.axis_index("c")
    s = lax.axis_index("s")
    r0 = pl.multiple_of(s * ROWS_PT, 8)

    @pl.when(c == 0)
    def _():
        pltpu.sync_copy(init0.at[pl.ds(r0, ROWS_PT)], acc.at[pl.ds(r0, ROWS_PT)])

    @pl.when(c == 1)
    def _():
        pltpu.sync_copy(init1.at[pl.ds(r0, ROWS_PT)], acc.at[pl.ds(r0, ROWS_PT)])

    plsc.subcore_barrier()

    base = s * CHUNKS_PER_TILE
    NCHT = CHUNKS_PER_TILE

    def prefetch_ids(ci, r):
        pltpu.async_copy(ids2.at[ci], ids_r[r], isem_r[r])
        pltpu.async_copy(vals2.at[ci], vals_r[r], isem_r[r])

    def wait_ids(r):
        pltpu.make_async_copy(ids2.at[0], ids_r[r], isem_r[r]).wait()
        pltpu.make_async_copy(vals2.at[0], vals_r[r], isem_r[r]).wait()

    def fire_gather(r):
        @pl.when(c == 0)
        def _():
            for j in range(JB):
                pltpu.async_copy(tbl0.at[ids_r[r].at[JB + j]],
                                 gbuf_r[r].at[pl.ds(j * 128, 128)], gsem_r[r])

        @pl.when(c == 1)
        def _():
            for j in range(JB):
                pltpu.async_copy(tbl1.at[ids_r[r].at[JB + j]],
                                 gbuf_r[r].at[pl.ds(j * 128, 128)], gsem_r[r])

    def drain_gather(r):
        for j in range(JB):
            pltpu.make_async_copy(tbl0.at[ids_r[r].at[JB + j]],
                                  gbuf_r[r].at[pl.ds(j * 128, 128)], gsem_r[r]).wait()

    def copy_rows(r):
        for t in range(8 * JB):
            srows_r[r][t // 8, pl.ds((t % 8) * 16, 16)] = (
                ids_r[r][t // 8, pl.ds((t % 8) * 16, 16)])

    def fire_scatter(r):
        for j in range(JB):
            pltpu.async_copy(gbuf_r[r].at[pl.ds(j * 128, 128)],
                             acc.at[srows_r[r].at[j]], ssem_r[r], add=True)

    def drain_scatter(r):
        for j in range(JB):
            pltpu.make_async_copy(gbuf_r[r].at[pl.ds(j * 128, 128)],
                                  acc.at[srows_r[r].at[j]], ssem_r[r]).wait()

    def scale(r):
        gbuf = gbuf_r[r]
        vals = vals_r[r]

        def scale16(g, carry2):
            e0 = g * 16
            v16 = vals[pl.ds(e0, 16)]
            for l in range(16):
                v = v16[l]
                e = e0 + l
                x0 = gbuf[e, pl.ds(0, 16)]
                x1 = gbuf[e, pl.ds(16, 16)]
                gbuf[e, pl.ds(0, 16)] = x0 * v
                gbuf[e, pl.ds(16, 16)] = x1 * v
            return carry2

        lax.fori_loop(0, CH // 16, scale16, 0)

    pltpu.sync_copy(ids2.at[base], ids_r[0])
    pltpu.sync_copy(vals2.at[base], vals_r[0])
    pltpu.sync_copy(ids2.at[base + 1], ids_r[1])
    pltpu.sync_copy(vals2.at[base + 1], vals_r[1])
    fire_gather(0)
    fire_gather(1)
    prefetch_ids(base + 2, 2)
    prefetch_ids(base + 3, 3)

    def trip_body(i, carry):
        for r in range(RING):
            j = RING * i + r
            drain_gather(r)
            copy_rows(r)
            scale(r)
            fire_scatter(r)

            @pl.when(j + RING < NCHT)
            def _():
                prefetch_ids(base + j + RING, r)

            @pl.when(j + 2 < NCHT)
            def _():
                wait_ids((r + 2) % RING)

            @pl.when(j >= 2)
            def _():
                drain_scatter((r + 2) % RING)

            @pl.when(j + 2 < NCHT)
            def _():
                fire_gather((r + 2) % RING)
        return carry

    lax.fori_loop(0, NTRIP, trip_body, 0)
    drain_scatter((CHUNKS_PER_TILE - 2) % RING)
    drain_scatter((CHUNKS_PER_TILE - 1) % RING)
    plsc.subcore_barrier()

    @pl.when(c == 0)
    def _():
        pltpu.sync_copy(acc.at[pl.ds(r0, ROWS_PT)], out0.at[pl.ds(r0, ROWS_PT)])

    @pl.when(c == 1)
    def _():
        pltpu.sync_copy(acc.at[pl.ds(r0, ROWS_PT)], out1.at[pl.ds(r0, ROWS_PT)])


@jax.jit
def _spmm(tbl0, tbl1, init0, init1, ids2, vals2):
    mesh = plsc.VectorSubcoreMesh(core_axis_name="c", subcore_axis_name="s")
    f = pl.kernel(
        _spmm_body,
        out_type=(jax.ShapeDtypeStruct((N_ROWS_PAD, HALF), jnp.float32),
                  jax.ShapeDtypeStruct((N_ROWS_PAD, HALF), jnp.float32)),
        mesh=mesh,
        scratch_types=[
            pltpu.VMEM_SHARED((N_ROWS_PAD, HALF), jnp.float32),
            [pltpu.VMEM((2 * JB, 128), jnp.int32) for _ in range(RING)],
            [pltpu.VMEM((CH,), jnp.float32) for _ in range(RING)],
            [pltpu.VMEM((JB, 128), jnp.int32) for _ in range(RING)],
            [pltpu.VMEM((CH, HALF), jnp.float32) for _ in range(RING)],
            [pltpu.SemaphoreType.DMA for _ in range(RING)],
            [pltpu.SemaphoreType.DMA for _ in range(RING)],
            [pltpu.SemaphoreType.DMA for _ in range(RING)],
        ],
        compiler_params=pltpu.CompilerParams(use_tc_tiling_on_sc=False),
    )
    return f(tbl0, tbl1, init0, init1, ids2, vals2)


def _normalize(x, eps=1e-12):
    n = jnp.sqrt(jnp.sum(x * x, axis=-1, keepdims=True))
    return x / jnp.maximum(n, eps)


def _ypred_kernel(u_ref, p_ref, o_ref):
    o_ref[...] = jnp.dot(u_ref[...], p_ref[...].T,
                         preferred_element_type=jnp.float32)


def kernel(E_u_0, E_i_0, adj_vals, ut, vt, u_mul_s, v_mul_s, adj_rows, adj_cols, users, pos_items, neg_items):
    neg = neg_items[:, 0]
    iids = jnp.concatenate([pos_items, neg], axis=0)

    pad = NNZ_PAD - NNZ
    rows_p = jnp.concatenate([adj_rows, jnp.zeros((pad,), jnp.int32)])
    cols_p = jnp.concatenate([adj_cols, jnp.zeros((pad,), jnp.int32)])
    vals_p = jnp.concatenate([adj_vals, jnp.zeros((pad,), jnp.float32)])
    ids_uv = jnp.concatenate([rows_p.reshape(N_CHUNKS, CH),
                              cols_p.reshape(N_CHUNKS, CH)],
                             axis=1).reshape(N_CHUNKS, 2 * JB, 128)
    ids_iv = jnp.concatenate([cols_p.reshape(N_CHUNKS, CH),
                              rows_p.reshape(N_CHUNKS, CH)],
                             axis=1).reshape(N_CHUNKS, 2 * JB, 128)
    vals2 = vals_p.reshape(N_CHUNKS, CH)

    rpad = N_ROWS_PAD - N_USERS
    Eu0p = jnp.concatenate([E_u_0, jnp.zeros((rpad, DIM), jnp.float32)])
    Ei0p = jnp.concatenate([E_i_0, jnp.zeros((rpad, DIM), jnp.float32)])
    Eu0a, Eu0b = Eu0p[:, :HALF], Eu0p[:, HALF:]
    Ei0a, Ei0b = Ei0p[:, :HALF], Ei0p[:, HALF:]

    su1a, su1b = _spmm(Ei0a[:N_ITEMS], Ei0b[:N_ITEMS], Eu0a, Eu0b, ids_uv, vals2)
    si1a, si1b = _spmm(Eu0a[:N_USERS], Eu0b[:N_USERS], Ei0a, Ei0b, ids_iv, vals2)
    eua, eub = _spmm(si1a[:N_ITEMS], si1b[:N_ITEMS], Eu0a, Eu0b, ids_uv, vals2)
    eia, eib = _spmm(su1a[:N_USERS], su1b[:N_USERS], Ei0a, Ei0b, ids_iv, vals2)

    E_u = jnp.concatenate([eua[:N_USERS], eub[:N_USERS]], axis=1)
    E_i = jnp.concatenate([eia[:N_ITEMS], eib[:N_ITEMS]], axis=1)
    S_u1 = jnp.concatenate([su1a[:N_USERS], su1b[:N_USERS]], axis=1)
    S_i1 = jnp.concatenate([si1a[:N_ITEMS], si1b[:N_ITEMS]], axis=1)

    vt_ei = vt @ S_i1
    ut_eu = ut @ S_u1
    G_u_b = E_u_0[users] + u_mul_s[users] @ vt_ei
    G_i_b = E_i_0[iids] + v_mul_s[iids] @ ut_eu

    neg_score = jnp.log(jnp.exp(G_u_b @ E_u.T / TEMP).sum(1) + 1e-08).mean()
    neg_score = neg_score + jnp.log(jnp.exp(G_i_b @ E_i.T / TEMP).sum(1) + 1e-08).mean()
    pos_score = jnp.clip((G_u_b * E_u[users]).sum(1) / TEMP, -5.0, 5.0).mean() \
        + jnp.clip((G_i_b * E_i[iids]).sum(1) / TEMP, -5.0, 5.0).mean()
    loss_s = -pos_score + neg_score

    u_e = _normalize(E_u[users])
    pos_e = _normalize(E_i[pos_items])
    y_pred = pl.pallas_call(
        _ypred_kernel,
        out_shape=jax.ShapeDtypeStruct((BATCH, BATCH), jnp.float32),
    )(u_e, pos_e)
    idx = jnp.arange(BATCH)
    diag = jnp.diagonal(y_pred)
    col0 = y_pred[:, 0]
    y2 = y_pred.at[idx, idx].set(col0)
    y2 = y2.at[idx, 0].set(diag)
    pos_logits = y2[:, 0] / T2
    loss_r = jnp.mean(-pos_logits + jax.nn.logsumexp(y2 / T3, axis=1))
    loss_reg = L2 * (jnp.sum(E_u_0 ** 2) + jnp.sum(E_i_0 ** 2))
    loss = loss_r + W1 * loss_s + loss_reg
    return (loss, loss_r, W1 * loss_s)


# speedup vs baseline: 1.1972x; 1.1972x over previous
import functools

import jax
import jax.numpy as jnp
from jax import lax
from jax.experimental import pallas as pl
from jax.experimental.pallas import tpu as pltpu
from jax.experimental.pallas import tpu_sc as plsc

N_USERS = 50000
N_ITEMS = 50000
DIM = 64
NNZ = 1600000
Q = 5
BATCH = 1024
TEMP = 0.2
T2 = 0.1
T3 = 1.0
W1 = 0.2
L2 = 1e-05

HALF = DIM // 2
NS = 16
CH = 256
JB = CH // 128
NNZ_PAD = 1609728
N_CHUNKS = NNZ_PAD // CH
CHUNKS_PER_TILE = N_CHUNKS // NS
NTRIP = CHUNKS_PER_TILE // 3
N_ROWS_PAD = 50048
ROWS_PT = N_ROWS_PAD // NS


def _spmm_body(tbl0, tbl1, init0, init1, ids2, vals2,
               out0, out1, acc,
               ids_r, vals_r, srows_r, gbuf_r,
               gsem_r, ssem_r, isem_r):
    c = lax.axis_index("c")
    s = lax.axis_index("s")
    r0 = pl.multiple_of(s * ROWS_PT, 8)

    @pl.when(c == 0)
    def _():
        pltpu.sync_copy(init0.at[pl.ds(r0, ROWS_PT)], acc.at[pl.ds(r0, ROWS_PT)])

    @pl.when(c == 1)
    def _():
        pltpu.sync_copy(init1.at[pl.ds(r0, ROWS_PT)], acc.at[pl.ds(r0, ROWS_PT)])

    plsc.subcore_barrier()

    base = s * CHUNKS_PER_TILE
    NCHT = CHUNKS_PER_TILE

    def prefetch_ids(ci, r):
        pltpu.async_copy(ids2.at[ci], ids_r[r], isem_r[r])
        pltpu.async_copy(vals2.at[ci], vals_r[r], isem_r[r])

    def wait_ids(r):
        pltpu.make_async_copy(ids2.at[0], ids_r[r], isem_r[r]).wait()
        pltpu.make_async_copy(vals2.at[0], vals_r[r], isem_r[r]).wait()

    def fire_gather(r):
        @pl.when(c == 0)
        def _():
            for j in range(JB):
                pltpu.async_copy(tbl0.at[ids_r[r].at[JB + j]],
                                 gbuf_r[r].at[pl.ds(j * 128, 128)], gsem_r[r])

        @pl.when(c == 1)
        def _():
            for j in range(JB):
                pltpu.async_copy(tbl1.at[ids_r[r].at[JB + j]],
                                 gbuf_r[r].at[pl.ds(j * 128, 128)], gsem_r[r])

    def drain_gather(r):
        for j in range(JB):
            pltpu.make_async_copy(tbl0.at[ids_r[r].at[JB + j]],
                                  gbuf_r[r].at[pl.ds(j * 128, 128)], gsem_r[r]).wait()

    def copy_rows(r):
        for t in range(8 * JB):
            srows_r[r][t // 8, pl.ds((t % 8) * 16, 16)] = (
                ids_r[r][t // 8, pl.ds((t % 8) * 16, 16)])

    def fire_scatter(r):
        for j in range(JB):
            pltpu.async_copy(gbuf_r[r].at[pl.ds(j * 128, 128)],
                             acc.at[srows_r[r].at[j]], ssem_r[r], add=True)

    def drain_scatter(r):
        for j in range(JB):
            pltpu.make_async_copy(gbuf_r[r].at[pl.ds(j * 128, 128)],
                                  acc.at[srows_r[r].at[j]], ssem_r[r]).wait()

    def scale(r):
        gbuf = gbuf_r[r]
        vals = vals_r[r]

        def scale16(g, carry2):
            e0 = g * 16
            v16 = vals[pl.ds(e0, 16)]
            for l in range(16):
                v = v16[l]
                e = e0 + l
                x0 = gbuf[e, pl.ds(0, 16)]
                x1 = gbuf[e, pl.ds(16, 16)]
                gbuf[e, pl.ds(0, 16)] = x0 * v
                gbuf[e, pl.ds(16, 16)] = x1 * v
            return carry2

        lax.fori_loop(0, CH // 16, scale16, 0)

    pltpu.sync_copy(ids2.at[base], ids_r[0])
    pltpu.sync_copy(vals2.at[base], vals_r[0])
    pltpu.sync_copy(ids2.at[base + 1], ids_r[1])
    pltpu.sync_copy(vals2.at[base + 1], vals_r[1])
    fire_gather(0)
    fire_gather(1)
    prefetch_ids(base + 2, 2)

    def trip_body(i, carry):
        for r in range(3):
            j = 3 * i + r
            drain_gather(r)
            copy_rows(r)
            scale(r)
            fire_scatter(r)

            @pl.when(j + 3 < NCHT)
            def _():
                prefetch_ids(base + j + 3, r)

            @pl.when(j + 2 < NCHT)
            def _():
                wait_ids((r + 2) % 3)

            @pl.when(j > 0)
            def _():
                drain_scatter((r + 2) % 3)

            @pl.when(j + 2 < NCHT)
            def _():
                fire_gather((r + 2) % 3)
        return carry

    lax.fori_loop(0, NTRIP, trip_body, 0)
    drain_scatter((CHUNKS_PER_TILE - 1) % 3)
    plsc.subcore_barrier()

    @pl.when(c == 0)
    def _():
        pltpu.sync_copy(acc.at[pl.ds(r0, ROWS_PT)], out0.at[pl.ds(r0, ROWS_PT)])

    @pl.when(c == 1)
    def _():
        pltpu.sync_copy(acc.at[pl.ds(r0, ROWS_PT)], out1.at[pl.ds(r0, ROWS_PT)])


@jax.jit
def _spmm(tbl0, tbl1, init0, init1, ids2, vals2):
    mesh = plsc.VectorSubcoreMesh(core_axis_name="c", subcore_axis_name="s")
    f = pl.kernel(
        _spmm_body,
        out_type=(jax.ShapeDtypeStruct((N_ROWS_PAD, HALF), jnp.float32),
                  jax.ShapeDtypeStruct((N_ROWS_PAD, HALF), jnp.float32)),
        mesh=mesh,
        scratch_types=[
            pltpu.VMEM_SHARED((N_ROWS_PAD, HALF), jnp.float32),
            [pltpu.VMEM((2 * JB, 128), jnp.int32) for _ in range(3)],
            [pltpu.VMEM((CH,), jnp.float32) for _ in range(3)],
            [pltpu.VMEM((JB, 128), jnp.int32) for _ in range(3)],
            [pltpu.VMEM((CH, HALF), jnp.float32) for _ in range(3)],
            [pltpu.SemaphoreType.DMA for _ in range(3)],
            [pltpu.SemaphoreType.DMA for _ in range(3)],
            [pltpu.SemaphoreType.DMA for _ in range(3)],
        ],
        compiler_params=pltpu.CompilerParams(use_tc_tiling_on_sc=False),
    )
    return f(tbl0, tbl1, init0, init1, ids2, vals2)


def _normalize(x, eps=1e-12):
    n = jnp.sqrt(jnp.sum(x * x, axis=-1, keepdims=True))
    return x / jnp.maximum(n, eps)


RB = 2000
N_RB = N_USERS // RB


def _loss_body(eua, eub, eia, eib, eu0, ei0, gub, gib, euu, eii, eip,
               lo, lr, ls, susum, sisum, regacc):
    k = pl.program_id(0)

    @pl.when(k == 0)
    def _():
        susum[...] = jnp.zeros_like(susum)
        sisum[...] = jnp.zeros_like(sisum)
        regacc[...] = jnp.zeros_like(regacc)

    eu_blk = jnp.concatenate([eua[...], eub[...]], axis=1)
    ei_blk = jnp.concatenate([eia[...], eib[...]], axis=1)
    lu = jnp.dot(gub[...], eu_blk.T, preferred_element_type=jnp.float32)
    li = jnp.dot(gib[...], ei_blk.T, preferred_element_type=jnp.float32)
    susum[...] += jnp.sum(jnp.exp(lu * (1.0 / TEMP)), axis=1, keepdims=True)
    sisum[...] += jnp.sum(jnp.exp(li * (1.0 / TEMP)), axis=1, keepdims=True)
    regacc[...] += (jnp.sum(eu0[...] * eu0[...])
                    + jnp.sum(ei0[...] * ei0[...]))[None, None]

    @pl.when(k == N_RB - 1)
    def _():
        neg_score = (jnp.mean(jnp.log(susum[...] + 1e-08))
                     + jnp.mean(jnp.log(sisum[...] + 1e-08)))
        pos_score = (
            jnp.mean(jnp.clip(jnp.sum(gub[...] * euu[...], axis=1) / TEMP,
                              -5.0, 5.0))
            + jnp.mean(jnp.clip(jnp.sum(gib[...] * eii[...], axis=1) / TEMP,
                                -5.0, 5.0)))
        loss_s = -pos_score + neg_score

        u_e = _normalize(euu[...])
        pos_e = _normalize(eip[...])
        y = jnp.dot(u_e, pos_e.T, preferred_element_type=jnp.float32)
        col = lax.broadcasted_iota(jnp.int32, (BATCH, BATCH), 1)
        row = lax.broadcasted_iota(jnp.int32, (BATCH, BATCH), 0)
        diag = jnp.sum(jnp.where(col == row, y, 0.0), axis=1, keepdims=True)
        col0 = y[:, 0:1]
        y2 = jnp.where(col == 0, diag, jnp.where(col == row, col0, y))
        lse = jnp.log(jnp.sum(jnp.exp(y2 * (1.0 / T3)), axis=1))
        loss_r = jnp.mean(-diag[:, 0] / T2 + lse)
        loss_reg = L2 * regacc[0, 0]
        lr[...] = jnp.reshape(loss_r, (1, 1))
        ls[...] = jnp.reshape(W1 * loss_s, (1, 1))
        lo[...] = jnp.reshape(loss_r + W1 * loss_s + loss_reg, (1, 1))


@jax.jit
def _loss_tc(eua, eub, eia, eib, eu0, ei0, gub, gib, euu, eii, eip):
    blk = lambda w: pl.BlockSpec((RB, w), lambda k: (k, 0))
    full = lambda a: pl.BlockSpec(a.shape, lambda k: (0,) * a.ndim)
    one = pl.BlockSpec((1, 1), lambda k: (0, 0))
    return pl.pallas_call(
        _loss_body,
        grid=(N_RB,),
        in_specs=[blk(HALF), blk(HALF), blk(HALF), blk(HALF),
                  blk(DIM), blk(DIM),
                  full(gub), full(gib), full(euu), full(eii), full(eip)],
        out_specs=[one, one, one],
        out_shape=[jax.ShapeDtypeStruct((1, 1), jnp.float32)] * 3,
        scratch_shapes=[
            pltpu.VMEM((BATCH, 1), jnp.float32),
            pltpu.VMEM((2 * BATCH, 1), jnp.float32),
            pltpu.VMEM((1, 1), jnp.float32),
        ],
    )(eua, eub, eia, eib, eu0, ei0, gub, gib, euu, eii, eip)


def kernel(E_u_0, E_i_0, adj_vals, ut, vt, u_mul_s, v_mul_s, adj_rows, adj_cols, users, pos_items, neg_items):
    neg = neg_items[:, 0]
    iids = jnp.concatenate([pos_items, neg], axis=0)

    pad = NNZ_PAD - NNZ
    rows_p = jnp.concatenate([adj_rows, jnp.zeros((pad,), jnp.int32)])
    cols_p = jnp.concatenate([adj_cols, jnp.zeros((pad,), jnp.int32)])
    vals_p = jnp.concatenate([adj_vals, jnp.zeros((pad,), jnp.float32)])
    ids_uv = jnp.concatenate([rows_p.reshape(N_CHUNKS, CH),
                              cols_p.reshape(N_CHUNKS, CH)],
                             axis=1).reshape(N_CHUNKS, 2 * JB, 128)
    ids_iv = jnp.concatenate([cols_p.reshape(N_CHUNKS, CH),
                              rows_p.reshape(N_CHUNKS, CH)],
                             axis=1).reshape(N_CHUNKS, 2 * JB, 128)
    vals2 = vals_p.reshape(N_CHUNKS, CH)

    rpad = N_ROWS_PAD - N_USERS
    Eu0p = jnp.concatenate([E_u_0, jnp.zeros((rpad, DIM), jnp.float32)])
    Ei0p = jnp.concatenate([E_i_0, jnp.zeros((rpad, DIM), jnp.float32)])
    Eu0a, Eu0b = Eu0p[:, :HALF], Eu0p[:, HALF:]
    Ei0a, Ei0b = Ei0p[:, :HALF], Ei0p[:, HALF:]

    su1a, su1b = _spmm(Ei0a[:N_ITEMS], Ei0b[:N_ITEMS], Eu0a, Eu0b, ids_uv, vals2)
    si1a, si1b = _spmm(Eu0a[:N_USERS], Eu0b[:N_USERS], Ei0a, Ei0b, ids_iv, vals2)
    eua, eub = _spmm(si1a[:N_ITEMS], si1b[:N_ITEMS], Eu0a, Eu0b, ids_uv, vals2)
    eia, eib = _spmm(su1a[:N_USERS], su1b[:N_USERS], Ei0a, Ei0b, ids_iv, vals2)

    vt_ei = jnp.concatenate([vt @ si1a[:N_ITEMS], vt @ si1b[:N_ITEMS]], axis=1)
    ut_eu = jnp.concatenate([ut @ su1a[:N_USERS], ut @ su1b[:N_USERS]], axis=1)
    G_u_b = E_u_0[users] + u_mul_s[users] @ vt_ei
    G_i_b = E_i_0[iids] + v_mul_s[iids] @ ut_eu

    E_u_users = jnp.concatenate([eua[users], eub[users]], axis=1)
    E_i_iids = jnp.concatenate([eia[iids], eib[iids]], axis=1)
    E_i_pos = jnp.concatenate([eia[pos_items], eib[pos_items]], axis=1)

    lo, lr, ls = _loss_tc(eua[:N_USERS], eub[:N_USERS],
                          eia[:N_ITEMS], eib[:N_ITEMS],
                          E_u_0, E_i_0, G_u_b, G_i_b,
                          E_u_users, E_i_iids, E_i_pos)
    return (lo[0, 0], lr[0, 0], ls[0, 0])


# speedup vs baseline: 1.2104x; 1.0110x over previous
import functools

import jax
import jax.numpy as jnp
from jax import lax
from jax.experimental import pallas as pl
from jax.experimental.pallas import tpu as pltpu
from jax.experimental.pallas import tpu_sc as plsc

N_USERS = 50000
N_ITEMS = 50000
DIM = 64
NNZ = 1600000
Q = 5
BATCH = 1024
TEMP = 0.2
T2 = 0.1
T3 = 1.0
W1 = 0.2
L2 = 1e-05

HALF = DIM // 2
NS = 16
CH = 256
JB = CH // 128
NNZ_PAD = 1609728
N_CHUNKS = NNZ_PAD // CH
CHUNKS_PER_TILE = N_CHUNKS // NS
NTRIP = CHUNKS_PER_TILE // 3
N_ROWS_PAD = 50048
ROWS_PT = N_ROWS_PAD // NS


def _spmm_body(tbl0, tbl1, init0, init1, rows2, cols2, vals2,
               out0, out1, acc,
               cols_r, rows_r, vals_r, gbuf_r,
               gsem_r, ssem_r, isem_r, rsem_r):
    c = lax.axis_index("c")
    s = lax.axis_index("s")
    r0 = pl.multiple_of(s * ROWS_PT, 8)

    @pl.when(c == 0)
    def _():
        pltpu.sync_copy(init0.at[pl.ds(r0, ROWS_PT)], acc.at[pl.ds(r0, ROWS_PT)])

    @pl.when(c == 1)
    def _():
        pltpu.sync_copy(init1.at[pl.ds(r0, ROWS_PT)], acc.at[pl.ds(r0, ROWS_PT)])

    plsc.subcore_barrier()

    base = s * CHUNKS_PER_TILE
    NCHT = CHUNKS_PER_TILE

    def prefetch_ids(ci, r):
        pltpu.async_copy(cols2.at[ci], cols_r[r], isem_r[r])
        pltpu.async_copy(vals2.at[ci], vals_r[r], isem_r[r])

    def wait_ids(r):
        pltpu.make_async_copy(cols2.at[0], cols_r[r], isem_r[r]).wait()
        pltpu.make_async_copy(vals2.at[0], vals_r[r], isem_r[r]).wait()

    def prefetch_rows(ci, r):
        pltpu.async_copy(rows2.at[ci], rows_r[r], rsem_r[r])

    def wait_rows(r):
        pltpu.make_async_copy(rows2.at[0], rows_r[r], rsem_r[r]).wait()

    def fire_gather(r):
        @pl.when(c == 0)
        def _():
            for j in range(JB):
                pltpu.async_copy(tbl0.at[cols_r[r].at[j]],
                                 gbuf_r[r].at[pl.ds(j * 128, 128)], gsem_r[r])

        @pl.when(c == 1)
        def _():
            for j in range(JB):
                pltpu.async_copy(tbl1.at[cols_r[r].at[j]],
                                 gbuf_r[r].at[pl.ds(j * 128, 128)], gsem_r[r])

    def drain_gather(r):
        for j in range(JB):
            pltpu.make_async_copy(tbl0.at[cols_r[r].at[j]],
                                  gbuf_r[r].at[pl.ds(j * 128, 128)], gsem_r[r]).wait()

    def fire_scatter(r):
        for j in range(JB):
            pltpu.async_copy(gbuf_r[r].at[pl.ds(j * 128, 128)],
                             acc.at[rows_r[r].at[j]], ssem_r[r], add=True)

    def drain_scatter(r):
        for j in range(JB):
            pltpu.make_async_copy(gbuf_r[r].at[pl.ds(j * 128, 128)],
                                  acc.at[rows_r[r].at[j]], ssem_r[r]).wait()

    def scale(r):
        gbuf = gbuf_r[r]
        vals = vals_r[r]

        def scale16(g, carry2):
            e0 = g * 16
            v16 = vals[pl.ds(e0, 16)]
            for l in range(16):
                v = v16[l]
                e = e0 + l
                x0 = gbuf[e, pl.ds(0, 16)]
                x1 = gbuf[e, pl.ds(16, 16)]
                gbuf[e, pl.ds(0, 16)] = x0 * v
                gbuf[e, pl.ds(16, 16)] = x1 * v
            return carry2

        lax.fori_loop(0, CH // 16, scale16, 0)

    pltpu.sync_copy(cols2.at[base], cols_r[0])
    pltpu.sync_copy(vals2.at[base], vals_r[0])
    pltpu.sync_copy(cols2.at[base + 1], cols_r[1])
    pltpu.sync_copy(vals2.at[base + 1], vals_r[1])
    fire_gather(0)
    fire_gather(1)
    prefetch_ids(base + 2, 2)
    prefetch_rows(base, 0)
    prefetch_rows(base + 1, 1)

    def trip_body(i, carry):
        for r in range(3):
            j = 3 * i + r
            drain_gather(r)
            scale(r)
            wait_rows(r)
            fire_scatter(r)

            @pl.when(j + 3 < NCHT)
            def _():
                prefetch_ids(base + j + 3, r)

            @pl.when(j + 2 < NCHT)
            def _():
                wait_ids((r + 2) % 3)

            @pl.when(j > 0)
            def _():
                drain_scatter((r + 2) % 3)

            @pl.when(j + 2 < NCHT)
            def _():
                prefetch_rows(base + j + 2, (r + 2) % 3)
                fire_gather((r + 2) % 3)
        return carry

    lax.fori_loop(0, NTRIP, trip_body, 0)
    drain_scatter((CHUNKS_PER_TILE - 1) % 3)
    plsc.subcore_barrier()

    @pl.when(c == 0)
    def _():
        pltpu.sync_copy(acc.at[pl.ds(r0, ROWS_PT)], out0.at[pl.ds(r0, ROWS_PT)])

    @pl.when(c == 1)
    def _():
        pltpu.sync_copy(acc.at[pl.ds(r0, ROWS_PT)], out1.at[pl.ds(r0, ROWS_PT)])


@jax.jit
def _spmm(tbl0, tbl1, init0, init1, rows2, cols2, vals2):
    mesh = plsc.VectorSubcoreMesh(core_axis_name="c", subcore_axis_name="s")
    f = pl.kernel(
        _spmm_body,
        out_type=(jax.ShapeDtypeStruct((N_ROWS_PAD, HALF), jnp.float32),
                  jax.ShapeDtypeStruct((N_ROWS_PAD, HALF), jnp.float32)),
        mesh=mesh,
        scratch_types=[
            pltpu.VMEM_SHARED((N_ROWS_PAD, HALF), jnp.float32),
            [pltpu.VMEM((JB, 128), jnp.int32) for _ in range(3)],
            [pltpu.VMEM((JB, 128), jnp.int32) for _ in range(3)],
            [pltpu.VMEM((CH,), jnp.float32) for _ in range(3)],
            [pltpu.VMEM((CH, HALF), jnp.float32) for _ in range(3)],
            [pltpu.SemaphoreType.DMA for _ in range(3)],
            [pltpu.SemaphoreType.DMA for _ in range(3)],
            [pltpu.SemaphoreType.DMA for _ in range(3)],
            [pltpu.SemaphoreType.DMA for _ in range(3)],
        ],
        compiler_params=pltpu.CompilerParams(use_tc_tiling_on_sc=False),
    )
    return f(tbl0, tbl1, init0, init1, rows2, cols2, vals2)


def _normalize(x, eps=1e-12):
    n = jnp.sqrt(jnp.sum(x * x, axis=-1, keepdims=True))
    return x / jnp.maximum(n, eps)


RB = 2000
N_RB = N_USERS // RB


def _loss_body(eua, eub, eia, eib, eu0, ei0, gub, gib, euu, eii, eip,
               lo, lr, ls, susum, sisum, regacc):
    k = pl.program_id(0)

    @pl.when(k == 0)
    def _():
        susum[...] = jnp.zeros_like(susum)
        sisum[...] = jnp.zeros_like(sisum)
        regacc[...] = jnp.zeros_like(regacc)

    eu_blk = jnp.concatenate([eua[...], eub[...]], axis=1)
    ei_blk = jnp.concatenate([eia[...], eib[...]], axis=1)
    lu = jnp.dot(gub[...].astype(jnp.bfloat16), eu_blk.T.astype(jnp.bfloat16),
                 preferred_element_type=jnp.float32)
    li = jnp.dot(gib[...].astype(jnp.bfloat16), ei_blk.T.astype(jnp.bfloat16),
                 preferred_element_type=jnp.float32)
    susum[...] += jnp.sum(jnp.exp(lu * (1.0 / TEMP)), axis=1, keepdims=True)
    sisum[...] += jnp.sum(jnp.exp(li * (1.0 / TEMP)), axis=1, keepdims=True)
    regacc[...] += (jnp.sum(eu0[...] * eu0[...])
                    + jnp.sum(ei0[...] * ei0[...]))[None, None]

    @pl.when(k == N_RB - 1)
    def _():
        neg_score = (jnp.mean(jnp.log(susum[...] + 1e-08))
                     + jnp.mean(jnp.log(sisum[...] + 1e-08)))
        pos_score = (
            jnp.mean(jnp.clip(jnp.sum(gub[...] * euu[...], axis=1) / TEMP,
                              -5.0, 5.0))
            + jnp.mean(jnp.clip(jnp.sum(gib[...] * eii[...], axis=1) / TEMP,
                                -5.0, 5.0)))
        loss_s = -pos_score + neg_score

        u_e = _normalize(euu[...])
        pos_e = _normalize(eip[...])
        y = jnp.dot(u_e, pos_e.T, preferred_element_type=jnp.float32)
        col = lax.broadcasted_iota(jnp.int32, (BATCH, BATCH), 1)
        row = lax.broadcasted_iota(jnp.int32, (BATCH, BATCH), 0)
        diag = jnp.sum(jnp.where(col == row, y, 0.0), axis=1, keepdims=True)
        col0 = y[:, 0:1]
        y2 = jnp.where(col == 0, diag, jnp.where(col == row, col0, y))
        lse = jnp.log(jnp.sum(jnp.exp(y2 * (1.0 / T3)), axis=1))
        loss_r = jnp.mean(-diag[:, 0] / T2 + lse)
        loss_reg = L2 * regacc[0, 0]
        lr[...] = jnp.reshape(loss_r, (1, 1))
        ls[...] = jnp.reshape(W1 * loss_s, (1, 1))
        lo[...] = jnp.reshape(loss_r + W1 * loss_s + loss_reg, (1, 1))


@jax.jit
def _loss_tc(eua, eub, eia, eib, eu0, ei0, gub, gib, euu, eii, eip):
    blk = lambda w: pl.BlockSpec((RB, w), lambda k: (k, 0))
    full = lambda a: pl.BlockSpec(a.shape, lambda k: (0,) * a.ndim)
    one = pl.BlockSpec((1, 1), lambda k: (0, 0))
    return pl.pallas_call(
        _loss_body,
        grid=(N_RB,),
        in_specs=[blk(HALF), blk(HALF), blk(HALF), blk(HALF),
                  blk(DIM), blk(DIM),
                  full(gub), full(gib), full(euu), full(eii), full(eip)],
        out_specs=[one, one, one],
        out_shape=[jax.ShapeDtypeStruct((1, 1), jnp.float32)] * 3,
        scratch_shapes=[
            pltpu.VMEM((BATCH, 1), jnp.float32),
            pltpu.VMEM((2 * BATCH, 1), jnp.float32),
            pltpu.VMEM((1, 1), jnp.float32),
        ],
    )(eua, eub, eia, eib, eu0, ei0, gub, gib, euu, eii, eip)


def kernel(E_u_0, E_i_0, adj_vals, ut, vt, u_mul_s, v_mul_s, adj_rows, adj_cols, users, pos_items, neg_items):
    neg = neg_items[:, 0]
    iids = jnp.concatenate([pos_items, neg], axis=0)

    pad = NNZ_PAD - NNZ
    rows_p = jnp.concatenate([adj_rows, jnp.zeros((pad,), jnp.int32)])
    cols_p = jnp.concatenate([adj_cols, jnp.zeros((pad,), jnp.int32)])
    vals_p = jnp.concatenate([adj_vals, jnp.zeros((pad,), jnp.float32)])
    rows3 = rows_p.reshape(N_CHUNKS, JB, 128)
    cols3 = cols_p.reshape(N_CHUNKS, JB, 128)
    vals2 = vals_p.reshape(N_CHUNKS, CH)

    rpad = N_ROWS_PAD - N_USERS
    Eu0p = jnp.concatenate([E_u_0, jnp.zeros((rpad, DIM), jnp.float32)])
    Ei0p = jnp.concatenate([E_i_0, jnp.zeros((rpad, DIM), jnp.float32)])
    Eu0a, Eu0b = Eu0p[:, :HALF], Eu0p[:, HALF:]
    Ei0a, Ei0b = Ei0p[:, :HALF], Ei0p[:, HALF:]

    su1a, su1b = _spmm(Ei0a[:N_ITEMS], Ei0b[:N_ITEMS], Eu0a, Eu0b, rows3, cols3, vals2)
    si1a, si1b = _spmm(Eu0a[:N_USERS], Eu0b[:N_USERS], Ei0a, Ei0b, cols3, rows3, vals2)
    eua, eub = _spmm(si1a[:N_ITEMS], si1b[:N_ITEMS], Eu0a, Eu0b, rows3, cols3, vals2)
    eia, eib = _spmm(su1a[:N_USERS], su1b[:N_USERS], Ei0a, Ei0b, cols3, rows3, vals2)

    vt_ei = jnp.concatenate([vt @ si1a[:N_ITEMS], vt @ si1b[:N_ITEMS]], axis=1)
    ut_eu = jnp.concatenate([ut @ su1a[:N_USERS], ut @ su1b[:N_USERS]], axis=1)
    G_u_b = E_u_0[users] + u_mul_s[users] @ vt_ei
    G_i_b = E_i_0[iids] + v_mul_s[iids] @ ut_eu

    E_u_users = jnp.concatenate([eua[users], eub[users]], axis=1)
    E_i_iids = jnp.concatenate([eia[iids], eib[iids]], axis=1)
    E_i_pos = jnp.concatenate([eia[pos_items], eib[pos_items]], axis=1)

    lo, lr, ls = _loss_tc(eua[:N_USERS], eub[:N_USERS],
                          eia[:N_ITEMS], eib[:N_ITEMS],
                          E_u_0, E_i_0, G_u_b, G_i_b,
                          E_u_users, E_i_iids, E_i_pos)
    return (lo[0, 0], lr[0, 0], ls[0, 0])
